# Initial kernel scaffold; baseline (speedup 1.0000x reference)
import jax
import jax.numpy as jnp
from jax.experimental import pallas as pl


def kernel(x, edge_vals, theta, bias, edge_index):
    raise NotImplementedError("write your pallas kernel here")


# speedup vs baseline: 2.4077x; 2.4077x over previous
import functools

import jax
import jax.numpy as jnp
from jax import lax
from jax.experimental import pallas as pl
from jax.experimental.pallas import tpu as pltpu
from jax.experimental.pallas import tpu_sc as plsc

_M = 10000
_MP = 10240
_E = 320000
_C = 128
_HC = 64
_NS = 16
_NC = 2
_EP = _E // _NS
_CH = 80
_NCH = _EP // _CH
_MT = _MP // _NS
_BM = _M
_LANES = 16


def _transpose_body(x_ref, o_ref):
    xb = x_ref[0]
    t = xb.T
    o_ref[0, :_M] = t[:, :_HC]
    o_ref[1, :_M] = t[:, _HC:]


def _sc_spmm2_body(x0f, cols_h, rows_h, vals_h, x1f, x2f,
                   colv, rowv, valv, gath, zb, y1s, y2s, sem):
    c = lax.axis_index("c")
    s = lax.axis_index("s")
    row0 = s * _MT
    coff = c * _MP

    zero = jnp.zeros((_LANES,), jnp.float32)

    def zrow(i, carry):
        for j in range(_HC // _LANES):
            zb[i, pl.ds(j * _LANES, _LANES)] = zero
        return carry

    lax.fori_loop(0, _MT, zrow, 0)
    pltpu.sync_copy(zb, y1s.at[pl.ds(row0, _MT)])
    pltpu.sync_copy(zb, y2s.at[pl.ds(row0, _MT)])
    plsc.subcore_barrier()

    def hop(src_tab, ysh):
        def chunk(i, carry):
            base = s * _EP + i * _CH
            pltpu.sync_copy(cols_h.at[pl.ds(base, _CH)], colv)
            pltpu.sync_copy(rows_h.at[pl.ds(base, _CH)], rowv)
            pltpu.sync_copy(vals_h.at[pl.ds(base, _CH)], valv)
            for j in range(_CH // _LANES):
                colv[pl.ds(j * _LANES, _LANES)] = (
                    colv[pl.ds(j * _LANES, _LANES)] + coff)
            pltpu.async_copy(src_tab.at[colv], gath, sem).wait()
            for g in range(_CH // _LANES):
                vv = valv[pl.ds(g * _LANES, _LANES)]
                for l in range(_LANES):
                    e = g * _LANES + l
                    bc = lax.gather(
                        vv, jnp.full((_LANES, 1), l, jnp.int32),
                        lax.GatherDimensionNumbers(
                            offset_dims=(), collapsed_slice_dims=(0,),
                            start_index_map=(0,)),
                        slice_sizes=(1,),
                        mode=lax.GatherScatterMode.PROMISE_IN_BOUNDS)
                    for j in range(_HC // _LANES):
                        sl = pl.ds(j * _LANES, _LANES)
                        gath[e, sl] = gath[e, sl] * bc
            pltpu.sync_copy(gath, ysh.at[rowv], add=True)
            return carry
        lax.fori_loop(0, _NCH, chunk, 0)

    hop(x0f, y1s)
    plsc.subcore_barrier()
    pltpu.sync_copy(y1s.at[pl.ds(row0, _MT)], x1f.at[pl.ds(coff + row0, _MT)])
    plsc.subcore_barrier()
    hop(x1f, y2s)
    plsc.subcore_barrier()
    pltpu.sync_copy(y2s.at[pl.ds(row0, _MT)], x2f.at[pl.ds(coff + row0, _MT)])


def _assemble_body(x_ref, x1a_ref, x1b_ref, x2a_ref, x2b_ref,
                   th_ref, bias_ref, o_ref):
    t0 = th_ref[0]
    t1 = th_ref[1]
    t2 = th_ref[2]
    xb = x_ref[0]
    dn = (((1,), (1,)), ((), ()))
    y = jnp.dot(t0 - t2, xb, preferred_element_type=jnp.float32)
    y += lax.dot_general(t1[:, :_HC], x1a_ref[...], dn,
                         preferred_element_type=jnp.float32)
    y += lax.dot_general(t1[:, _HC:], x1b_ref[...], dn,
                         preferred_element_type=jnp.float32)
    y += lax.dot_general(2.0 * t2[:, :_HC], x2a_ref[...], dn,
                         preferred_element_type=jnp.float32)
    y += lax.dot_general(2.0 * t2[:, _HC:], x2b_ref[...], dn,
                         preferred_element_type=jnp.float32)
    o_ref[0] = y + bias_ref[...]


@jax.jit
def kernel(x, edge_vals, theta, bias, edge_index):
    f32 = jnp.float32

    x0h = pl.pallas_call(
        _transpose_body,
        grid=(1,),
        in_specs=[pl.BlockSpec((1, _C, _M), lambda j: (0, 0, 0))],
        out_specs=pl.BlockSpec((_NC, _MP, _HC), lambda j: (0, 0, 0)),
        out_shape=jax.ShapeDtypeStruct((_NC, _MP, _HC), f32),
    )(x)
    x0f = x0h.reshape(_NC * _MP, _HC)

    mesh = plsc.VectorSubcoreMesh(core_axis_name="c", subcore_axis_name="s")
    x1f, x2f = pl.kernel(
        _sc_spmm2_body,
        out_type=[jax.ShapeDtypeStruct((_NC * _MP, _HC), f32),
                  jax.ShapeDtypeStruct((_NC * _MP, _HC), f32)],
        mesh=mesh,
        scratch_types=[
            pltpu.VMEM((_CH,), jnp.int32),
            pltpu.VMEM((_CH,), jnp.int32),
            pltpu.VMEM((_CH,), f32),
            pltpu.VMEM((_CH, _HC), f32),
            pltpu.VMEM((_MT, _HC), f32),
            pltpu.VMEM_SHARED((_MP, _HC), f32),
            pltpu.VMEM_SHARED((_MP, _HC), f32),
            pltpu.SemaphoreType.DMA,
        ],
        compiler_params=pltpu.CompilerParams(use_tc_tiling_on_sc=False),
    )(x0f, edge_index[1], edge_index[0], edge_vals)

    thT = jnp.transpose(theta, (2, 0, 1))
    bias2 = bias.reshape(_C, 1)
    y = pl.pallas_call(
        _assemble_body,
        grid=(_M // _BM,),
        in_specs=[
            pl.BlockSpec((1, _C, _BM), lambda j: (0, 0, j)),
            pl.BlockSpec((_BM, _HC), lambda j: (j, 0)),
            pl.BlockSpec((_BM, _HC), lambda j: (j, 0)),
            pl.BlockSpec((_BM, _HC), lambda j: (j, 0)),
            pl.BlockSpec((_BM, _HC), lambda j: (j, 0)),
            pl.BlockSpec((3, _C, _C), lambda j: (0, 0, 0)),
            pl.BlockSpec((_C, 1), lambda j: (0, 0)),
        ],
        out_specs=pl.BlockSpec((1, _C, _BM), lambda j: (0, 0, j)),
        out_shape=jax.ShapeDtypeStruct((1, _C, _M), f32),
    )(x, x1f[:_M], x1f[_MP:_MP + _M], x2f[:_M], x2f[_MP:_MP + _M],
      thT, bias2)
    return y


# speedup vs baseline: 5.7583x; 2.3916x over previous
import functools

import jax
import jax.numpy as jnp
from jax import lax
from jax.experimental import pallas as pl
from jax.experimental.pallas import tpu as pltpu
from jax.experimental.pallas import tpu_sc as plsc

_M = 10000
_MP = 10240
_E = 320000
_C = 128
_HC = 64
_NS = 16
_NC = 2
_EP = _E // _NS
_CH = 80
_NCH = _EP // _CH
_GRP = 5
_NSTG = 2
_SCH = _NCH // _NSTG
_MT = _MP // _NS
_ZR = 80
_BM = _M
_LANES = 16


def _transpose_body(x_ref, o_ref):
    xb = x_ref[0]
    t = xb.T
    o_ref[0, :_M] = t[:, :_HC]
    o_ref[1, :_M] = t[:, _HC:]


def _bcast_lane(vv, l):
    return lax.gather(
        vv, jnp.full((_LANES, 1), l, jnp.int32),
        lax.GatherDimensionNumbers(
            offset_dims=(), collapsed_slice_dims=(0,),
            start_index_map=(0,)),
        slice_sizes=(1,),
        mode=lax.GatherScatterMode.PROMISE_IN_BOUNDS)


def _sc_spmm2_body(x0h, cols3, rows3, vals3, x1h, x2h,
                   colv, rowv, valv, gath, zb, ysh, gsem, ssem):
    c = lax.axis_index("c")
    s = lax.axis_index("s")
    row0 = s * _MT

    zero = jnp.zeros((_LANES,), jnp.float32)

    def zrow(i, carry):
        for j in range(_HC // _LANES):
            zb[i, pl.ds(j * _LANES, _LANES)] = zero
        return carry

    lax.fori_loop(0, _ZR, zrow, 0)

    def zero_acc():
        for t in range(_MT // _ZR):
            pltpu.sync_copy(zb, ysh.at[pl.ds(row0 + t * _ZR, _ZR)])

    zero_acc()

    plsc.subcore_barrier()

    def hop(src_tab, acc):
        for h in range(_NSTG):
            st = pl.ds(h * _SCH, _SCH)
            pltpu.sync_copy(cols3.at[s, st], colv)
            pltpu.sync_copy(rows3.at[s, st], rowv)
            pltpu.sync_copy(vals3.at[s, st], valv)

            def group(g, carry):
                i0 = g * _GRP
                for j in range(_GRP):
                    pltpu.async_copy(src_tab.at[colv.at[i0 + j]],
                                     gath.at[j], gsem.at[j])
                for j in range(_GRP):
                    pltpu.make_async_copy(src_tab.at[colv.at[i0 + j]],
                                          gath.at[j], gsem.at[j]).wait()
                    for gg in range(_CH // _LANES):
                        vv = valv[i0 + j, pl.ds(gg * _LANES, _LANES)]
                        for l in range(_LANES):
                            e = gg * _LANES + l
                            bc = _bcast_lane(vv, l)
                            for jj in range(_HC // _LANES):
                                sl = pl.ds(jj * _LANES, _LANES)
                                gath[j, e, sl] = gath[j, e, sl] * bc
                    pltpu.async_copy(gath.at[j], acc.at[rowv.at[i0 + j]],
                                     ssem.at[j], add=True)
                for j in range(_GRP):
                    pltpu.make_async_copy(gath.at[j],
                                          acc.at[rowv.at[i0 + j]],
                                          ssem.at[j]).wait()
                return carry
            lax.fori_loop(0, _SCH // _GRP, group, 0)

    hop(x0h.at[c], ysh)
    plsc.subcore_barrier()
    pltpu.sync_copy(ysh.at[pl.ds(row0, _MT)], x1h.at[c, pl.ds(row0, _MT)])
    plsc.subcore_barrier()
    zero_acc()
    plsc.subcore_barrier()
    hop(x1h.at[c], ysh)
    plsc.subcore_barrier()
    pltpu.sync_copy(ysh.at[pl.ds(row0, _MT)], x2h.at[c, pl.ds(row0, _MT)])


def _assemble_body(x_ref, x1a_ref, x1b_ref, x2a_ref, x2b_ref,
                   th_ref, bias_ref, o_ref):
    t0 = th_ref[0]
    t1 = th_ref[1]
    t2 = th_ref[2]
    xb = x_ref[0]
    dn = (((1,), (1,)), ((), ()))
    y = jnp.dot(t0 - t2, xb, preferred_element_type=jnp.float32)
    y += lax.dot_general(t1[:, :_HC], x1a_ref[...], dn,
                         preferred_element_type=jnp.float32)
    y += lax.dot_general(t1[:, _HC:], x1b_ref[...], dn,
                         preferred_element_type=jnp.float32)
    y += lax.dot_general(2.0 * t2[:, :_HC], x2a_ref[...], dn,
                         preferred_element_type=jnp.float32)
    y += lax.dot_general(2.0 * t2[:, _HC:], x2b_ref[...], dn,
                         preferred_element_type=jnp.float32)
    o_ref[0] = y + bias_ref[...]


@jax.jit
def kernel(x, edge_vals, theta, bias, edge_index):
    f32 = jnp.float32

    x0h = pl.pallas_call(
        _transpose_body,
        grid=(1,),
        in_specs=[pl.BlockSpec((1, _C, _M), lambda j: (0, 0, 0))],
        out_specs=pl.BlockSpec((_NC, _MP, _HC), lambda j: (0, 0, 0)),
        out_shape=jax.ShapeDtypeStruct((_NC, _MP, _HC), f32),
    )(x)
    cols3 = edge_index[1].reshape(_NS, _NCH, _CH)
    rows3 = edge_index[0].reshape(_NS, _NCH, _CH)
    vals3 = edge_vals.reshape(_NS, _NCH, _CH)

    mesh = plsc.VectorSubcoreMesh(core_axis_name="c", subcore_axis_name="s")
    x1h, x2h = pl.kernel(
        _sc_spmm2_body,
        out_type=[jax.ShapeDtypeStruct((_NC, _MP, _HC), f32),
                  jax.ShapeDtypeStruct((_NC, _MP, _HC), f32)],
        mesh=mesh,
        scratch_types=[
            pltpu.VMEM((_SCH, _CH), jnp.int32),
            pltpu.VMEM((_SCH, _CH), jnp.int32),
            pltpu.VMEM((_SCH, _CH), f32),
            pltpu.VMEM((_GRP, _CH, _HC), f32),
            pltpu.VMEM((_ZR, _HC), f32),
            pltpu.VMEM_SHARED((_MP, _HC), f32),
            pltpu.SemaphoreType.DMA((_GRP,)),
            pltpu.SemaphoreType.DMA((_GRP,)),
        ],
        compiler_params=pltpu.CompilerParams(use_tc_tiling_on_sc=False),
    )(x0h, cols3, rows3, vals3)

    thT = jnp.transpose(theta, (2, 0, 1))
    bias2 = bias.reshape(_C, 1)
    y = pl.pallas_call(
        _assemble_body,
        grid=(_M // _BM,),
        in_specs=[
            pl.BlockSpec((1, _C, _BM), lambda j: (0, 0, j)),
            pl.BlockSpec((_BM, _HC), lambda j: (j, 0)),
            pl.BlockSpec((_BM, _HC), lambda j: (j, 0)),
            pl.BlockSpec((_BM, _HC), lambda j: (j, 0)),
            pl.BlockSpec((_BM, _HC), lambda j: (j, 0)),
            pl.BlockSpec((3, _C, _C), lambda j: (0, 0, 0)),
            pl.BlockSpec((_C, 1), lambda j: (0, 0)),
        ],
        out_specs=pl.BlockSpec((1, _C, _BM), lambda j: (0, 0, j)),
        out_shape=jax.ShapeDtypeStruct((1, _C, _M), f32),
    )(x, x1h[0, :_M], x1h[1, :_M], x2h[0, :_M], x2h[1, :_M], thT, bias2)
    return y
